# baseline (speedup 1.0000x reference)
import functools

import jax
import jax.numpy as jnp
from jax import lax
from jax.experimental import pallas as pl
from jax.experimental.pallas import tpu as pltpu
from jax.experimental.pallas import tpu_sc as plsc

BATCH = 4096
NUM_WORDS = 50
EMB = 32
LAT = 64
HALF = 16

NC = 2
NS = 16
NW = NC * NS
B_PER_W = BATCH // NW
CHUNK = 16
NCHUNK = B_PER_W // CHUNK
ROWS = CHUNK * NUM_WORDS

_mesh = plsc.VectorSubcoreMesh(core_axis_name="c", subcore_axis_name="s")


KG = 10
NKG = NUM_WORDS // KG


@functools.partial(
    pl.kernel,
    mesh=_mesh,
    compiler_params=pltpu.CompilerParams(use_tc_tiling_on_sc=False),
    out_type=jax.ShapeDtypeStruct((BATCH, EMB), jnp.float32),
    scratch_types=[
        pltpu.VMEM((NUM_WORDS * B_PER_W,), jnp.int32),
        pltpu.VMEM((KG * B_PER_W, EMB), jnp.float32),
        pltpu.VMEM((KG * B_PER_W, EMB), jnp.float32),
        pltpu.VMEM((B_PER_W, EMB), jnp.float32),
        pltpu.SemaphoreType.DMA,
        pltpu.SemaphoreType.DMA,
        pltpu.SemaphoreType.DMA,
    ],
)
def _sc_pool(wT_hbm, table_hbm, out_hbm, idxk_v, buf0, buf1, acc_v,
             sem0, sem1, sem_fill):
    wid = lax.axis_index("s") * NC + lax.axis_index("c")
    obase = wid * B_PER_W

    fills = []
    for k in range(NUM_WORDS):
        fills.append(pltpu.async_copy(
            wT_hbm.at[k, pl.ds(obase, B_PER_W)],
            idxk_v.at[pl.ds(k * B_PER_W, B_PER_W)],
            sem_fill))
    for f in fills:
        f.wait()

    bufs = (buf0, buf1)
    sems = (sem0, sem1)
    handles = [None, None]

    def start(g):
        idx_sl = idxk_v.at[pl.ds(g * KG * B_PER_W, KG * B_PER_W)]
        handles[g % 2] = pltpu.async_copy(
            table_hbm.at[idx_sl], bufs[g % 2], sems[g % 2])

    def process(g):
        buf = bufs[g % 2]
        first = g == 0

        def row_body(r, carry):
            for h in range(2):
                sl = pl.ds(h * HALF, HALF)
                b = [buf[j * B_PER_W + r, sl] for j in range(KG)]
                s = (((b[0] + b[1]) + (b[2] + b[3]))
                     + ((b[4] + b[5]) + (b[6] + b[7]))) + (b[8] + b[9])
                if first:
                    acc_v[r, sl] = s
                else:
                    acc_v[r, sl] = acc_v[r, sl] + s
            return carry

        lax.fori_loop(0, B_PER_W, row_body, 0)

    start(0)
    for g in range(1, NKG):
        start(g)
        handles[(g - 1) % 2].wait()
        process(g - 1)
    handles[(NKG - 1) % 2].wait()
    process(NKG - 1)

    scale = jnp.float32(1.0 / NUM_WORDS)

    def scale_body(r, carry):
        for h in range(2):
            sl = pl.ds(h * HALF, HALF)
            acc_v[r, sl] = acc_v[r, sl] * scale
        return carry

    lax.fori_loop(0, B_PER_W, scale_body, 0)
    pltpu.sync_copy(acc_v, out_hbm.at[pl.ds(obase, B_PER_W)])


def _heads_body(p_ref, wmu_ref, bmu_ref, wsig_ref, bsig_ref, mean_ref, logstd_ref):
    p = p_ref[...]
    mean_ref[...] = (
        jnp.dot(p, wmu_ref[...], preferred_element_type=jnp.float32)
        + bmu_ref[...]
    )
    logstd_ref[...] = (
        jnp.dot(p, wsig_ref[...], preferred_element_type=jnp.float32)
        + bsig_ref[...]
    )


_heads = pl.pallas_call(
    _heads_body,
    out_shape=(
        jax.ShapeDtypeStruct((BATCH, LAT), jnp.float32),
        jax.ShapeDtypeStruct((BATCH, LAT), jnp.float32),
    ),
)


def kernel(w, table, W_mu, b_mu, W_sig, b_sig):
    pooled = _sc_pool(w.T.astype(jnp.int32), table)
    mean, logstd = _heads(
        pooled, W_mu, b_mu.reshape(1, LAT), W_sig, b_sig.reshape(1, LAT))
    return (mean, logstd)
